# Initial kernel scaffold; baseline (speedup 1.0000x reference)
import jax
import jax.numpy as jnp
from jax.experimental import pallas as pl


def kernel(x, weight):
    raise NotImplementedError("write your pallas kernel here")


# speedup vs baseline: 10.8777x; 10.8777x over previous
import jax
import jax.numpy as jnp
from jax.experimental import pallas as pl
from jax.experimental.pallas import tpu as pltpu

_K = 1024
_N = 32768
_ROWS_PER_BLOCK = 8


def _body(x_ref, w_ref, o_ref):
    xw = x_ref[...] * w_ref[...][None, :]
    bits = jax.lax.bitcast_convert_type(xw, jnp.int32) & jnp.int32(0x7FFFFFFF)
    b = xw.shape[0]

    def bs_step(_, lohi):
        lo, hi = lohi
        mid = jax.lax.shift_right_logical(lo + hi, 1)
        cnt = jnp.sum((bits >= mid).astype(jnp.int32), axis=1, keepdims=True)
        ge = cnt >= _K
        return jnp.where(ge, mid, lo), jnp.where(ge, hi, mid)

    lo0 = jnp.zeros((b, 1), jnp.int32)
    hi0 = jnp.full((b, 1), jnp.int32(0x7F800000))
    t, _ = jax.lax.fori_loop(0, 31, bs_step, (lo0, hi0))

    gt = bits > t
    eq = bits == t
    cnt_gt = jnp.sum(gt.astype(jnp.int32), axis=1, keepdims=True)
    m = _K - cnt_gt

    col = jax.lax.broadcasted_iota(jnp.int32, xw.shape, 1)
    eq_i = eq.astype(jnp.int32)

    def idx_step(_, lohi):
        lo, hi = lohi
        mid = jax.lax.shift_right_arithmetic(lo + hi, 1)
        cnt = jnp.sum(jnp.where(col <= mid, eq_i, 0), axis=1, keepdims=True)
        ge = cnt >= m
        return jnp.where(ge, lo, mid), jnp.where(ge, mid, hi)

    ilo0 = jnp.full((b, 1), jnp.int32(-1))
    ihi0 = jnp.full((b, 1), jnp.int32(xw.shape[1] - 1))
    _, jm = jax.lax.fori_loop(0, 15, idx_step, (ilo0, ihi0))

    zero = gt | (eq & (col <= jm))
    o_ref[...] = jnp.where(zero, 0.0, xw)


def kernel(x, weight):
    grid = (x.shape[0] // _ROWS_PER_BLOCK,)
    return pl.pallas_call(
        _body,
        grid=grid,
        in_specs=[
            pl.BlockSpec((_ROWS_PER_BLOCK, _N), lambda i: (i, 0)),
            pl.BlockSpec((_N,), lambda i: (0,)),
        ],
        out_specs=pl.BlockSpec((_ROWS_PER_BLOCK, _N), lambda i: (i, 0)),
        out_shape=jax.ShapeDtypeStruct(x.shape, x.dtype),
    )(x, weight)
